# baseline (speedup 1.0000x reference)
import jax
import jax.numpy as jnp
import numpy as np
from jax import lax
from jax.experimental import pallas as pl
from jax.experimental.pallas import tpu as pltpu
from jax.experimental.pallas import tpu_sc as plsc

K = 64
ROWS = 128
N = 8192
NC = 2
NS = 16
NW = NC * NS
ROWS_PER_W = ROWS // NW
L = 16
NV = N // L
NACC = 8
INT_MIN = np.int32(-(2 ** 31))
NEG_INF = np.float32(-np.inf)


def _lane0(v):
    return jnp.squeeze(lax.slice(v, (0,), (1,)), 0)


def _popcnt(m):
    return _lane0(plsc.all_reduce_population_count(m))


def _keyvec(fv):
    ik = lax.bitcast_convert_type(fv, jnp.int32)
    return jnp.where(ik >= 0, ik, ik ^ jnp.int32(0x7FFFFFFF))


def _sc_body(x_hbm, mask_hbm, out_hbm, x_v0, x_v1, m_v0, m_v1, fv_v, cand_f,
             cand_k, cand_i, sel_i, o_v0, o_v1, semx, semy):
    wid = lax.axis_index("s") * NC + lax.axis_index("c")
    row0 = wid * ROWS_PER_W
    iota = lax.iota(jnp.int32, L)

    def one_row(xb, mb, ob):
        def p1(j, accs):
            accs = list(accs)
            for t in range(NACC):
                i = NACC * j + t
                xv = xb[pl.ds(i * L, L)]
                mv = mb[pl.ds(i * L, L)]
                fv = jnp.where(mv == 0, NEG_INF, xv)
                fv_v[pl.ds(i * L, L)] = fv
                accs[t] = jnp.maximum(accs[t], fv)
            return tuple(accs)

        init = tuple(
            jnp.full((L,), NEG_INF, jnp.float32) for _ in range(NACC))
        accs = lax.fori_loop(0, NV // NACC, p1, init)

        vmn, vmx = accs[0], accs[0]
        for t in range(1, NACC):
            vmn = jnp.minimum(vmn, accs[t])
            vmx = jnp.maximum(vmx, accs[t])
        tmin = -jnp.max(-vmn)
        tmax = jnp.max(vmx)
        lo_s = _lane0(_keyvec(jnp.full((L,), 0.0, jnp.float32) + tmin))
        hi_s = _lane0(_keyvec(jnp.full((L,), 0.0, jnp.float32) + tmax)) \
            + jnp.int32(1)

        def coarse(_s, c):
            lo, hi = c
            half = lax.shift_right_logical(hi - lo, 1)
            mid = lo + half
            mk = jnp.full((L,), 0, jnp.int32) + mid
            mf = lax.bitcast_convert_type(
                jnp.where(mk >= 0, mk, mk ^ jnp.int32(0x7FFFFFFF)),
                jnp.float32)
            cacc = jnp.zeros((L,), jnp.int32)
            for t in range(NACC):
                cacc = cacc + (accs[t] >= mf).astype(jnp.int32)
            ge = jnp.sum(cacc) >= K
            sel_mid = (half != 0) & ge
            sel_hi = (half != 0) & (~ge)
            return (jnp.where(sel_mid, mid, lo), jnp.where(sel_hi, mid, hi))

        lo0, _ = lax.fori_loop(0, 6, coarse, (lo_s, hi_s))
        hi0 = hi_s
        lk = jnp.full((L,), 0, jnp.int32) + lo0
        tlow0 = _lane0(lax.bitcast_convert_type(
            jnp.where(lk >= 0, lk, lk ^ jnp.int32(0x7FFFFFFF)), jnp.float32))

        def p2(j, off):
            fvs, selms, pcs = [], [], []
            for t in range(8):
                i = 8 * j + t
                fv = fv_v[pl.ds(i * L, L)]
                selm = fv >= tlow0
                fvs.append(fv)
                selms.append(selm)
                pcs.append(_popcnt(selm))
            offs = [off]
            for t in range(7):
                offs.append(offs[-1] + pcs[t])
            for t in range(8):
                i = 8 * j + t
                iv = iota + i * L
                plsc.store_compressed(
                    cand_f.at[pl.ds(offs[t], L)], fvs[t], mask=selms[t])
                plsc.store_compressed(
                    cand_i.at[pl.ds(offs[t], L)], iv, mask=selms[t])
            return offs[7] + pcs[7]

        nc = lax.fori_loop(0, NV // 8, p2, jnp.int32(0))
        nvc = (nc + L - 1) // L

        def p2b(i, _unused):
            fv = cand_f[pl.ds(i * L, L)]
            cand_k[pl.ds(i * L, L)] = _keyvec(fv)
            return 0

        lax.fori_loop(0, nvc, p2b, 0)
        cand_k[pl.ds(nc, L)] = jnp.full((L,), INT_MIN, jnp.int32)

        def bs_cond(c):
            lo, hi, _ = c
            span = hi - lo
            return (span != 0) & (span != 1)

        def bs_body(c):
            lo, hi, chi = c
            span = hi - lo
            h = lax.shift_right_logical(span, 1)
            q = lax.shift_right_logical(span, 2)
            p1_ = lo + q
            p2_ = lo + h
            p3_ = lo + h + q

            def cnt_body(i, a):
                a1, a2, a3 = a
                kv = cand_k[pl.ds(i * L, L)]
                return (a1 + (kv >= p1_).astype(jnp.int32),
                        a2 + (kv >= p2_).astype(jnp.int32),
                        a3 + (kv >= p3_).astype(jnp.int32))

            z = jnp.zeros((L,), jnp.int32)
            a1, a2, a3 = lax.fori_loop(0, nvc, cnt_body, (z, z, z))
            c1 = jnp.sum(a1)
            c2 = jnp.sum(a2)
            c3 = jnp.sum(a3)
            g1 = c1 >= K
            g2 = c2 >= K
            g3 = c3 >= K
            nlo = jnp.where(g3, p3_,
                            jnp.where(g2, p2_, jnp.where(g1, p1_, lo)))
            nhi = jnp.where(g3, hi,
                            jnp.where(g2, p3_, jnp.where(g1, p2_, p1_)))
            nchi = jnp.where(g3, chi,
                             jnp.where(g2, c3, jnp.where(g1, c2, c1)))
            return (nlo, nhi, nchi)

        thr, _, c_gt = lax.while_loop(bs_cond, bs_body,
                                      (lo0, hi0, jnp.int32(0)))
        slots = K - c_gt

        def p4(i, carry):
            off, eqs = carry
            kv = cand_k[pl.ds(i * L, L)]
            iv = cand_i[pl.ds(i * L, L)]
            gt = kv > thr
            eq = kv == thr
            eqc = plsc.cumsum(eq.astype(jnp.int32))
            sel = gt | (eq & (eqc + eqs <= slots))
            plsc.store_compressed(sel_i.at[pl.ds(off, L)], iv, mask=sel)
            return (off + _popcnt(sel), eqs + _popcnt(eq))

        lax.fori_loop(0, nvc, p4, (jnp.int32(0), jnp.int32(0)))

        for j in range(K // L):
            idx = sel_i[pl.ds(j * L, L)]
            ob[pl.ds(j * L, L)] = plsc.load_gather(xb, [idx])

    def do_pair(p, _unused):
        rowA = row0 + 2 * p
        rowB = rowA + 1
        hxA = pltpu.async_copy(x_hbm.at[rowA], x_v0, semx)
        hmA = pltpu.async_copy(mask_hbm.at[rowA], m_v0, semx)
        hxB = pltpu.async_copy(x_hbm.at[rowB], x_v1, semy)
        hmB = pltpu.async_copy(mask_hbm.at[rowB], m_v1, semy)
        hxA.wait()
        hmA.wait()
        one_row(x_v0, m_v0, o_v0)
        hoA = pltpu.async_copy(o_v0, out_hbm.at[rowA], semx)
        hxB.wait()
        hmB.wait()
        one_row(x_v1, m_v1, o_v1)
        hoB = pltpu.async_copy(o_v1, out_hbm.at[rowB], semy)
        hoA.wait()
        hoB.wait()
        return 0

    lax.fori_loop(0, ROWS_PER_W // 2, do_pair, 0)


@jax.jit
def _kmax_sc(x, mask):
    mesh = plsc.VectorSubcoreMesh(core_axis_name="c", subcore_axis_name="s")
    return pl.kernel(
        _sc_body,
        out_type=jax.ShapeDtypeStruct((ROWS, K), jnp.float32),
        mesh=mesh,
        compiler_params=pltpu.CompilerParams(needs_layout_passes=False),
        scratch_types=[
            pltpu.VMEM((N,), jnp.float32),
            pltpu.VMEM((N,), jnp.float32),
            pltpu.VMEM((N,), jnp.int32),
            pltpu.VMEM((N,), jnp.int32),
            pltpu.VMEM((N,), jnp.float32),
            pltpu.VMEM((N + L,), jnp.float32),
            pltpu.VMEM((N + L,), jnp.int32),
            pltpu.VMEM((N + L,), jnp.int32),
            pltpu.VMEM((K + L,), jnp.int32),
            pltpu.VMEM((K,), jnp.float32),
            pltpu.VMEM((K,), jnp.float32),
            pltpu.SemaphoreType.DMA,
            pltpu.SemaphoreType.DMA,
        ],
    )(x, mask)


def kernel(x, mask):
    return _kmax_sc(x, mask)


# speedup vs baseline: 1.0318x; 1.0318x over previous
import jax
import jax.numpy as jnp
import numpy as np
from jax import lax
from jax.experimental import pallas as pl
from jax.experimental.pallas import tpu as pltpu
from jax.experimental.pallas import tpu_sc as plsc

K = 64
ROWS = 128
N = 8192
NC = 2
NS = 16
NW = NC * NS
ROWS_PER_W = ROWS // NW
L = 16
NV = N // L
NACC = 8
INT_MIN = np.int32(-(2 ** 31))
NEG_INF = np.float32(-np.inf)


def _lane0(v):
    return jnp.squeeze(lax.slice(v, (0,), (1,)), 0)


def _popcnt(m):
    return _lane0(plsc.all_reduce_population_count(m))


def _keyvec(fv):
    ik = lax.bitcast_convert_type(fv, jnp.int32)
    return jnp.where(ik >= 0, ik, ik ^ jnp.int32(0x7FFFFFFF))


def _sc_body(x_hbm, mask_hbm, out_hbm, x_v0, x_v1, m_v0, m_v1, fv_v, cand_f,
             cand_k, cand_i, sel_i, o_v0, o_v1, o_v2, o_v3, semx, semy):
    wid = lax.axis_index("s") * NC + lax.axis_index("c")
    row0 = wid * ROWS_PER_W
    iota = lax.iota(jnp.int32, L)

    def one_row(xb, mb, ob):
        def p1(j, accs):
            accs = list(accs)
            for t in range(NACC):
                i = NACC * j + t
                xv = xb[pl.ds(i * L, L)]
                mv = mb[pl.ds(i * L, L)]
                fv = jnp.where(mv == 0, NEG_INF, xv)
                fv_v[pl.ds(i * L, L)] = fv
                accs[t] = jnp.maximum(accs[t], fv)
            return tuple(accs)

        init = tuple(
            jnp.full((L,), NEG_INF, jnp.float32) for _ in range(NACC))
        accs = lax.fori_loop(0, NV // NACC, p1, init)

        vmn, vmx = accs[0], accs[0]
        for t in range(1, NACC):
            vmn = jnp.minimum(vmn, accs[t])
            vmx = jnp.maximum(vmx, accs[t])
        tmin = -jnp.max(-vmn)
        tmax = jnp.max(vmx)
        lo_s = _lane0(_keyvec(jnp.full((L,), 0.0, jnp.float32) + tmin))
        hi_s = _lane0(_keyvec(jnp.full((L,), 0.0, jnp.float32) + tmax)) \
            + jnp.int32(1)

        def coarse(_s, c):
            lo, hi = c
            half = lax.shift_right_logical(hi - lo, 1)
            mid = lo + half
            mk = jnp.full((L,), 0, jnp.int32) + mid
            mf = lax.bitcast_convert_type(
                jnp.where(mk >= 0, mk, mk ^ jnp.int32(0x7FFFFFFF)),
                jnp.float32)
            cacc = jnp.zeros((L,), jnp.int32)
            for t in range(NACC):
                cacc = cacc + (accs[t] >= mf).astype(jnp.int32)
            ge = jnp.sum(cacc) >= K
            sel_mid = (half != 0) & ge
            sel_hi = (half != 0) & (~ge)
            return (jnp.where(sel_mid, mid, lo), jnp.where(sel_hi, mid, hi))

        lo0, _ = lax.fori_loop(0, 6, coarse, (lo_s, hi_s))
        hi0 = hi_s
        lk = jnp.full((L,), 0, jnp.int32) + lo0
        tlow0 = _lane0(lax.bitcast_convert_type(
            jnp.where(lk >= 0, lk, lk ^ jnp.int32(0x7FFFFFFF)), jnp.float32))

        def p2(j, off):
            fvs, selms, pcs = [], [], []
            for t in range(8):
                i = 8 * j + t
                fv = fv_v[pl.ds(i * L, L)]
                selm = fv >= tlow0
                fvs.append(fv)
                selms.append(selm)
                pcs.append(_popcnt(selm))
            offs = [off]
            for t in range(7):
                offs.append(offs[-1] + pcs[t])
            for t in range(8):
                i = 8 * j + t
                iv = iota + i * L
                plsc.store_compressed(
                    cand_f.at[pl.ds(offs[t], L)], fvs[t], mask=selms[t])
                plsc.store_compressed(
                    cand_i.at[pl.ds(offs[t], L)], iv, mask=selms[t])
            return offs[7] + pcs[7]

        nc = lax.fori_loop(0, NV // 8, p2, jnp.int32(0))
        nvc = (nc + L - 1) // L

        def p2b(i, _unused):
            fv = cand_f[pl.ds(i * L, L)]
            cand_k[pl.ds(i * L, L)] = _keyvec(fv)
            return 0

        lax.fori_loop(0, nvc, p2b, 0)
        cand_k[pl.ds(nc, L)] = jnp.full((L,), INT_MIN, jnp.int32)

        def bs_cond(c):
            lo, hi, _ = c
            span = hi - lo
            return (span != 0) & (span != 1)

        def bs_body(c):
            lo, hi, chi = c
            span = hi - lo
            h = lax.shift_right_logical(span, 1)
            q = lax.shift_right_logical(span, 2)
            p1_ = lo + q
            p2_ = lo + h
            p3_ = lo + h + q

            def cnt_body(i, a):
                a1, a2, a3 = a
                kv = cand_k[pl.ds(i * L, L)]
                return (a1 + (kv >= p1_).astype(jnp.int32),
                        a2 + (kv >= p2_).astype(jnp.int32),
                        a3 + (kv >= p3_).astype(jnp.int32))

            z = jnp.zeros((L,), jnp.int32)
            a1, a2, a3 = lax.fori_loop(0, nvc, cnt_body, (z, z, z))
            c1 = jnp.sum(a1)
            c2 = jnp.sum(a2)
            c3 = jnp.sum(a3)
            g1 = c1 >= K
            g2 = c2 >= K
            g3 = c3 >= K
            nlo = jnp.where(g3, p3_,
                            jnp.where(g2, p2_, jnp.where(g1, p1_, lo)))
            nhi = jnp.where(g3, hi,
                            jnp.where(g2, p3_, jnp.where(g1, p2_, p1_)))
            nchi = jnp.where(g3, chi,
                             jnp.where(g2, c3, jnp.where(g1, c2, c1)))
            return (nlo, nhi, nchi)

        thr, _, c_gt = lax.while_loop(bs_cond, bs_body,
                                      (lo0, hi0, jnp.int32(0)))
        slots = K - c_gt

        def p4(i, carry):
            off, eqs = carry
            kv = cand_k[pl.ds(i * L, L)]
            iv = cand_i[pl.ds(i * L, L)]
            gt = kv > thr
            eq = kv == thr
            eqc = plsc.cumsum(eq.astype(jnp.int32))
            sel = gt | (eq & (eqc + eqs <= slots))
            plsc.store_compressed(sel_i.at[pl.ds(off, L)], iv, mask=sel)
            return (off + _popcnt(sel), eqs + _popcnt(eq))

        lax.fori_loop(0, nvc, p4, (jnp.int32(0), jnp.int32(0)))

        for j in range(K // L):
            idx = sel_i[pl.ds(j * L, L)]
            ob[pl.ds(j * L, L)] = plsc.load_gather(xb, [idx])

    xbufs = (x_v0, x_v1)
    mbufs = (m_v0, m_v1)
    obufs = (o_v0, o_v1, o_v2, o_v3)

    def issue(r, b):
        sem = semx if b == 0 else semy
        hx = pltpu.async_copy(x_hbm.at[row0 + r], xbufs[b], sem)
        hm = pltpu.async_copy(mask_hbm.at[row0 + r], mbufs[b], sem)
        return hx, hm

    pending = issue(0, 0)
    out_handles = []
    for r in range(ROWS_PER_W):
        b = r % 2
        pending[0].wait()
        pending[1].wait()
        if r + 1 < ROWS_PER_W:
            pending = issue(r + 1, 1 - b)
        one_row(xbufs[b], mbufs[b], obufs[r])
        out_handles.append(
            pltpu.async_copy(obufs[r], out_hbm.at[row0 + r],
                             semx if b else semy))

    for h in out_handles:
        h.wait()


@jax.jit
def _kmax_sc(x, mask):
    mesh = plsc.VectorSubcoreMesh(core_axis_name="c", subcore_axis_name="s")
    return pl.kernel(
        _sc_body,
        out_type=jax.ShapeDtypeStruct((ROWS, K), jnp.float32),
        mesh=mesh,
        compiler_params=pltpu.CompilerParams(needs_layout_passes=False),
        scratch_types=[
            pltpu.VMEM((N,), jnp.float32),
            pltpu.VMEM((N,), jnp.float32),
            pltpu.VMEM((N,), jnp.int32),
            pltpu.VMEM((N,), jnp.int32),
            pltpu.VMEM((N,), jnp.float32),
            pltpu.VMEM((N + L,), jnp.float32),
            pltpu.VMEM((N + L,), jnp.int32),
            pltpu.VMEM((N + L,), jnp.int32),
            pltpu.VMEM((K + L,), jnp.int32),
            pltpu.VMEM((K,), jnp.float32),
            pltpu.VMEM((K,), jnp.float32),
            pltpu.VMEM((K,), jnp.float32),
            pltpu.VMEM((K,), jnp.float32),
            pltpu.SemaphoreType.DMA,
            pltpu.SemaphoreType.DMA,
        ],
    )(x, mask)


def kernel(x, mask):
    return _kmax_sc(x, mask)


# speedup vs baseline: 1.0600x; 1.0273x over previous
import jax
import jax.numpy as jnp
import numpy as np
from jax import lax
from jax.experimental import pallas as pl
from jax.experimental.pallas import tpu as pltpu
from jax.experimental.pallas import tpu_sc as plsc

K = 64
ROWS = 128
N = 8192
NC = 2
NS = 16
NW = NC * NS
ROWS_PER_W = ROWS // NW
L = 16
NV = N // L
NACC = 8
INT_MIN = np.int32(-(2 ** 31))
NEG_INF = np.float32(-np.inf)


def _lane0(v):
    return jnp.squeeze(lax.slice(v, (0,), (1,)), 0)


def _popcnt(m):
    return _lane0(plsc.all_reduce_population_count(m))


def _keyvec(fv):
    ik = lax.bitcast_convert_type(fv, jnp.int32)
    return jnp.where(ik >= 0, ik, ik ^ jnp.int32(0x7FFFFFFF))


def _sc_body(x_hbm, mask_hbm, out_hbm, x_v0, x_v1, m_v0, m_v1, fv_v,
             cand_k, cand_i, sel_i, o_v0, o_v1, o_v2, o_v3, semx, semy):
    wid = lax.axis_index("s") * NC + lax.axis_index("c")
    row0 = wid * ROWS_PER_W
    iota = lax.iota(jnp.int32, L)

    def one_row(xb, mb, ob):
        def p1(j, accs):
            accs = list(accs)
            for t in range(NACC):
                i = NACC * j + t
                xv = xb[pl.ds(i * L, L)]
                mv = mb[pl.ds(i * L, L)]
                fv = jnp.where(mv == 0, NEG_INF, xv)
                fv_v[pl.ds(i * L, L)] = fv
                accs[t] = jnp.maximum(accs[t], fv)
            return tuple(accs)

        init = tuple(
            jnp.full((L,), NEG_INF, jnp.float32) for _ in range(NACC))
        accs = lax.fori_loop(0, NV // NACC, p1, init)

        vmn, vmx = accs[0], accs[0]
        for t in range(1, NACC):
            vmn = jnp.minimum(vmn, accs[t])
            vmx = jnp.maximum(vmx, accs[t])
        tmin = -jnp.max(-vmn)
        tmax = jnp.max(vmx)
        lo_s = _lane0(_keyvec(jnp.full((L,), 0.0, jnp.float32) + tmin))
        hi_s = _lane0(_keyvec(jnp.full((L,), 0.0, jnp.float32) + tmax)) \
            + jnp.int32(1)

        def coarse(_s, c):
            lo, hi = c
            half = lax.shift_right_logical(hi - lo, 1)
            mid = lo + half
            mk = jnp.full((L,), 0, jnp.int32) + mid
            mf = lax.bitcast_convert_type(
                jnp.where(mk >= 0, mk, mk ^ jnp.int32(0x7FFFFFFF)),
                jnp.float32)
            cacc = jnp.zeros((L,), jnp.int32)
            for t in range(NACC):
                cacc = cacc + (accs[t] >= mf).astype(jnp.int32)
            ge = jnp.sum(cacc) >= K
            sel_mid = (half != 0) & ge
            sel_hi = (half != 0) & (~ge)
            return (jnp.where(sel_mid, mid, lo), jnp.where(sel_hi, mid, hi))

        lo0, _ = lax.fori_loop(0, 6, coarse, (lo_s, hi_s))
        hi0 = hi_s
        lk = jnp.full((L,), 0, jnp.int32) + lo0
        tlow0 = _lane0(lax.bitcast_convert_type(
            jnp.where(lk >= 0, lk, lk ^ jnp.int32(0x7FFFFFFF)), jnp.float32))

        def p2(j, off):
            selms, pcs = [], []
            for t in range(8):
                i = 8 * j + t
                fv = fv_v[pl.ds(i * L, L)]
                selm = fv >= tlow0
                selms.append(selm)
                pcs.append(_popcnt(selm))
            offs = [off]
            for t in range(7):
                offs.append(offs[-1] + pcs[t])
            for t in range(8):
                i = 8 * j + t
                iv = iota + i * L
                plsc.store_compressed(
                    cand_i.at[pl.ds(offs[t], L)], iv, mask=selms[t])
            return offs[7] + pcs[7]

        nc = lax.fori_loop(0, NV // 8, p2, jnp.int32(0))
        nvc = (nc + L - 1) // L
        cand_i[pl.ds(nc, L)] = iota

        def p2b(i, _unused):
            iv = cand_i[pl.ds(i * L, L)]
            fv = plsc.load_gather(fv_v, [iv])
            cand_k[pl.ds(i * L, L)] = _keyvec(fv)
            return 0

        lax.fori_loop(0, nvc, p2b, 0)
        cand_k[pl.ds(nc, L)] = jnp.full((L,), INT_MIN, jnp.int32)

        def bs_cond(c):
            lo, hi, _ = c
            span = hi - lo
            return (span != 0) & (span != 1)

        def bs_body(c):
            lo, hi, chi = c
            span = hi - lo
            h = lax.shift_right_logical(span, 1)
            q = lax.shift_right_logical(span, 2)
            p1_ = lo + q
            p2_ = lo + h
            p3_ = lo + h + q

            def cnt_body(i, a):
                a1, a2, a3 = a
                kv = cand_k[pl.ds(i * L, L)]
                return (a1 + (kv >= p1_).astype(jnp.int32),
                        a2 + (kv >= p2_).astype(jnp.int32),
                        a3 + (kv >= p3_).astype(jnp.int32))

            z = jnp.zeros((L,), jnp.int32)
            a1, a2, a3 = lax.fori_loop(0, nvc, cnt_body, (z, z, z))
            c1 = jnp.sum(a1)
            c2 = jnp.sum(a2)
            c3 = jnp.sum(a3)
            g1 = c1 >= K
            g2 = c2 >= K
            g3 = c3 >= K
            nlo = jnp.where(g3, p3_,
                            jnp.where(g2, p2_, jnp.where(g1, p1_, lo)))
            nhi = jnp.where(g3, hi,
                            jnp.where(g2, p3_, jnp.where(g1, p2_, p1_)))
            nchi = jnp.where(g3, chi,
                             jnp.where(g2, c3, jnp.where(g1, c2, c1)))
            return (nlo, nhi, nchi)

        thr, _, c_gt = lax.while_loop(bs_cond, bs_body,
                                      (lo0, hi0, jnp.int32(0)))
        slots = K - c_gt

        def p4(i, carry):
            off, eqs = carry
            kv = cand_k[pl.ds(i * L, L)]
            iv = cand_i[pl.ds(i * L, L)]
            gt = kv > thr
            eq = kv == thr
            eqc = plsc.cumsum(eq.astype(jnp.int32))
            sel = gt | (eq & (eqc + eqs <= slots))
            plsc.store_compressed(sel_i.at[pl.ds(off, L)], iv, mask=sel)
            return (off + _popcnt(sel), eqs + _popcnt(eq))

        lax.fori_loop(0, nvc, p4, (jnp.int32(0), jnp.int32(0)))

        for j in range(K // L):
            idx = sel_i[pl.ds(j * L, L)]
            ob[pl.ds(j * L, L)] = plsc.load_gather(xb, [idx])

    xbufs = (x_v0, x_v1)
    mbufs = (m_v0, m_v1)
    obufs = (o_v0, o_v1, o_v2, o_v3)

    def issue(r, b):
        sem = semx if b == 0 else semy
        hx = pltpu.async_copy(x_hbm.at[row0 + r], xbufs[b], sem)
        hm = pltpu.async_copy(mask_hbm.at[row0 + r], mbufs[b], sem)
        return hx, hm

    pending = issue(0, 0)
    out_handles = []
    for r in range(ROWS_PER_W):
        b = r % 2
        pending[0].wait()
        pending[1].wait()
        if r + 1 < ROWS_PER_W:
            pending = issue(r + 1, 1 - b)
        one_row(xbufs[b], mbufs[b], obufs[r])
        out_handles.append(
            pltpu.async_copy(obufs[r], out_hbm.at[row0 + r],
                             semx if b else semy))

    for h in out_handles:
        h.wait()


@jax.jit
def _kmax_sc(x, mask):
    mesh = plsc.VectorSubcoreMesh(core_axis_name="c", subcore_axis_name="s")
    return pl.kernel(
        _sc_body,
        out_type=jax.ShapeDtypeStruct((ROWS, K), jnp.float32),
        mesh=mesh,
        compiler_params=pltpu.CompilerParams(needs_layout_passes=False),
        scratch_types=[
            pltpu.VMEM((N,), jnp.float32),
            pltpu.VMEM((N,), jnp.float32),
            pltpu.VMEM((N,), jnp.int32),
            pltpu.VMEM((N,), jnp.int32),
            pltpu.VMEM((N,), jnp.float32),
            pltpu.VMEM((N + L,), jnp.int32),
            pltpu.VMEM((N + L,), jnp.int32),
            pltpu.VMEM((K + L,), jnp.int32),
            pltpu.VMEM((K,), jnp.float32),
            pltpu.VMEM((K,), jnp.float32),
            pltpu.VMEM((K,), jnp.float32),
            pltpu.VMEM((K,), jnp.float32),
            pltpu.SemaphoreType.DMA,
            pltpu.SemaphoreType.DMA,
        ],
    )(x, mask)


def kernel(x, mask):
    return _kmax_sc(x, mask)
